# baseline (speedup 1.0000x reference)
import functools

import jax
import jax.numpy as jnp
from jax import lax
from jax.experimental import pallas as pl
from jax.experimental.pallas import tpu as pltpu
from jax.experimental.pallas import tpu_sc as plsc

B = 16384
C = 1000
SMOOTH = 0.1
EPS = 1e-12
LN2 = 0.6931471805599453

B_TC = 12288
B_SC = B - B_TC
NSPLIT = 2
STEP_ROWS = 1024
GRID = B_TC // (STEP_ROWS * NSPLIT)
HALF = STEP_ROWS // 2

NW = 32
SC_ROWS = B_SC // NW
CH = 8
NCHUNK = SC_ROWS // CH
NSUPER = NCHUNK // 2
NVEC = (C // 16) & ~1
NPAIR = NVEC // 2
TAIL0 = C - 16
TAILN = C - NVEC * 16

P5 = (0.04392862784795337, -0.40947558576646115, 1.6101775468967987,
      -3.5202188381455293, 5.069756316633291, -2.7941536765360535)


def _log2(v):
    bits = plsc.bitcast(v, jnp.int32)
    e = (bits >> 23) - 127
    m = plsc.bitcast((bits & 0x007FFFFF) | 0x3F800000, jnp.float32)
    poly = jnp.float32(P5[0])
    for c in P5[1:]:
        poly = poly * m + jnp.float32(c)
    return e.astype(jnp.float32) + poly


def _pair_terms(va, vb, s1, s2):
    pa = jnp.maximum(va, EPS)
    pb = jnp.maximum(vb, EPS)
    s1 = s1 + _log2(pa * pb)
    s2 = s2 + _log2((1.0 - pa) * (1.0 - pb))
    return s1, s2


def _sc_half(x, labels_pad):
    mesh = plsc.VectorSubcoreMesh(core_axis_name="c", subcore_axis_name="s")

    @functools.partial(
        pl.kernel,
        mesh=mesh,
        compiler_params=pltpu.CompilerParams(needs_layout_passes=False,
                                             skip_device_barrier=True),
        out_type=jax.ShapeDtypeStruct((NW, 16), jnp.float32),
        scratch_types=[
            pltpu.VMEM((2, CH, C), jnp.float32),
            pltpu.VMEM((SC_ROWS + 16,), jnp.int32),
            pltpu.VMEM((16,), jnp.float32),
            pltpu.SemaphoreType.DMA,
            pltpu.SemaphoreType.DMA,
        ],
    )
    def k(x_hbm, lbl_hbm, o_hbm, bufs, lblbuf, vout, sem0, sem1):
        lane = lax.iota(jnp.int32, 16)
        wid = lax.axis_index("s") * 2 + lax.axis_index("c")
        r0 = B_TC + wid * SC_ROWS
        pltpu.sync_copy(lbl_hbm.at[pl.ds(r0, SC_ROWS + 16)], lblbuf)

        def chunk_copy(ch_idx, slot, sem):
            return pltpu.make_async_copy(
                x_hbm.at[pl.ds(r0 + ch_idx * CH, CH), :],
                bufs.at[slot], sem)

        chunk_copy(0, 0, sem0).start()

        def process(buf_slot, ch_idx, s1, s2, sg):
            for r in range(CH):
                def pair_body(kk, carry):
                    c1, c2 = carry
                    va = bufs[buf_slot, r, pl.ds(32 * kk, 16)]
                    vb = bufs[buf_slot, r, pl.ds(32 * kk + 16, 16)]
                    return _pair_terms(va, vb, c1, c2)

                s1, s2 = lax.fori_loop(0, NPAIR, pair_body, (s1, s2))
                vt = bufs[buf_slot, r, pl.ds(TAIL0, 16)]
                fresh = lane >= (16 - TAILN)
                pt = jnp.where(fresh, jnp.maximum(vt, EPS), 1.0)
                qt = jnp.where(fresh, 1.0 - jnp.maximum(vt, EPS), 1.0)
                s1 = s1 + _log2(pt)
                s2 = s2 + _log2(qt)
            cols = lblbuf[pl.ds(ch_idx * CH, 16)]
            rows = lane & (CH - 1)
            slot_idx = jnp.full((16,), buf_slot, jnp.int32)
            g = plsc.load_gather(bufs, [slot_idx, rows, cols])
            valid = lane < CH
            gv = jnp.where(valid, jnp.maximum(g, EPS), 0.5)
            sg = sg + _log2(gv) - _log2(1.0 - gv)
            return s1, s2, sg

        def super_body(i, carry):
            s1, s2, sg = carry
            chunk_copy(2 * i + 1, 1, sem1).start()
            chunk_copy(2 * i, 0, sem0).wait()
            s1, s2, sg = process(0, 2 * i, s1, s2, sg)

            @pl.when(i < NSUPER - 1)
            def _():
                chunk_copy(2 * i + 2, 0, sem0).start()

            chunk_copy(2 * i + 1, 1, sem1).wait()
            return process(1, 2 * i + 1, s1, s2, sg)

        zero = jnp.zeros((16,), jnp.float32)
        s1, s2, sg = lax.fori_loop(0, NSUPER, super_body, (zero, zero, zero))
        vout[...] = (SMOOTH * s1 + (1.0 - SMOOTH) * s2
                     + (1.0 - SMOOTH) * sg)
        pltpu.sync_copy(vout, o_hbm.at[wid])

    return k(x, labels_pad)


def _tc_body(*refs):
    x_refs = refs[:NSPLIT]
    l_refs = refs[NSPLIT:2 * NSPLIT]
    o_ref, acc_ref = refs[2 * NSPLIT], refs[2 * NSPLIT + 1]
    step = pl.program_id(0)

    @pl.when(step == 0)
    def _():
        acc_ref[0, 0] = 0.0

    s = 0.0
    for x_ref, l_ref in zip(x_refs, l_refs):
        x = x_ref[...]
        cols = lax.broadcasted_iota(jnp.int32, (STEP_ROWS, C), 1)
        m = cols == l_ref[...]
        g_row = jnp.sum(jnp.where(m, x, 0.0), axis=1, keepdims=True)
        g = jnp.clip(g_row, EPS, 1.0 - EPS)
        s += (1.0 - SMOOTH) * jnp.sum(jnp.log(g) - jnp.log(1.0 - g))
        pa = jnp.clip(x[:HALF], EPS, 1.0 - EPS)
        pb = jnp.clip(x[HALF:], EPS, 1.0 - EPS)
        s += SMOOTH * jnp.sum(jnp.log(pa * pb))
        s += (1.0 - SMOOTH) * jnp.sum(jnp.log((1.0 - pa) * (1.0 - pb)))
    acc_ref[0, 0] += s

    @pl.when(step == GRID - 1)
    def _():
        o_ref[0, 0] = acc_ref[0, 0]


def _combine_body(t_ref, s_ref, o_ref):
    o_ref[0, 0] = -(t_ref[0, 0] + LN2 * jnp.sum(s_ref[...])) * (1.0 / (B * C))


def kernel(inputs, outputs, labels):
    del inputs
    lab = labels.astype(jnp.int32)
    sc_part = _sc_half(outputs, jnp.concatenate(
        [lab, jnp.zeros((16,), jnp.int32)]))
    lab2d = lab.reshape(B, 1)
    tc_part = pl.pallas_call(
        _tc_body,
        grid=(GRID,),
        in_specs=[
            pl.BlockSpec((STEP_ROWS, C), lambda i, k=k: (NSPLIT * i + k, 0))
            for k in range(NSPLIT)
        ] + [
            pl.BlockSpec((STEP_ROWS, 1), lambda i, k=k: (NSPLIT * i + k, 0))
            for k in range(NSPLIT)
        ],
        out_specs=pl.BlockSpec((1, 1), lambda i: (0, 0),
                               memory_space=pltpu.SMEM),
        out_shape=jax.ShapeDtypeStruct((1, 1), jnp.float32),
        scratch_shapes=[pltpu.SMEM((1, 1), jnp.float32)],
    )(*([outputs] * NSPLIT + [lab2d] * NSPLIT))
    loss = pl.pallas_call(
        _combine_body,
        in_specs=[
            pl.BlockSpec(memory_space=pltpu.SMEM),
            pl.BlockSpec(memory_space=pltpu.VMEM),
        ],
        out_specs=pl.BlockSpec(memory_space=pltpu.SMEM),
        out_shape=jax.ShapeDtypeStruct((1, 1), jnp.float32),
    )(tc_part, sc_part)
    return loss[0, 0]


# speedup vs baseline: 1.2120x; 1.2120x over previous
import functools

import jax
import jax.numpy as jnp
from jax import lax
from jax.experimental import pallas as pl
from jax.experimental.pallas import tpu as pltpu
from jax.experimental.pallas import tpu_sc as plsc

B = 16384
C = 1000
SMOOTH = 0.1
EPS = 1e-12

NSPLIT = 2
STEP_ROWS = 1024
GRID = B // (STEP_ROWS * NSPLIT)
HALF = STEP_ROWS // 2


def _dense_body(*refs):
    x_refs = refs[:NSPLIT]
    l_refs = refs[NSPLIT:2 * NSPLIT]
    o_ref, acc_ref = refs[2 * NSPLIT], refs[2 * NSPLIT + 1]
    step = pl.program_id(0)

    @pl.when(step == 0)
    def _():
        acc_ref[0, 0] = 0.0

    s = 0.0
    for x_ref, l_ref in zip(x_refs, l_refs):
        x = x_ref[...]
        cols = lax.broadcasted_iota(jnp.int32, (STEP_ROWS, C), 1)
        m = cols == l_ref[...]
        g_row = jnp.sum(jnp.where(m, x, 0.0), axis=1, keepdims=True)
        g = jnp.clip(g_row, EPS, 1.0 - EPS)
        s += (1.0 - SMOOTH) * jnp.sum(jnp.log(g) - jnp.log(1.0 - g))
        pa = jnp.clip(x[:HALF], EPS, 1.0 - EPS)
        pb = jnp.clip(x[HALF:], EPS, 1.0 - EPS)
        s += SMOOTH * jnp.sum(jnp.log(pa * pb))
        s += (1.0 - SMOOTH) * jnp.sum(jnp.log((1.0 - pa) * (1.0 - pb)))
    acc_ref[0, 0] += s

    @pl.when(step == GRID - 1)
    def _():
        o_ref[0, 0] = -acc_ref[0, 0] * (1.0 / (B * C))


def kernel(inputs, outputs, labels):
    del inputs
    lab2d = labels.astype(jnp.int32).reshape(B, 1)
    loss = pl.pallas_call(
        _dense_body,
        grid=(GRID,),
        in_specs=[
            pl.BlockSpec((STEP_ROWS, C), lambda i, k=k: (NSPLIT * i + k, 0))
            for k in range(NSPLIT)
        ] + [
            pl.BlockSpec((STEP_ROWS, 1), lambda i, k=k: (NSPLIT * i + k, 0))
            for k in range(NSPLIT)
        ],
        out_specs=pl.BlockSpec((1, 1), lambda i: (0, 0),
                               memory_space=pltpu.SMEM),
        out_shape=jax.ShapeDtypeStruct((1, 1), jnp.float32),
        scratch_shapes=[pltpu.SMEM((1, 1), jnp.float32)],
    )(*([outputs] * NSPLIT + [lab2d] * NSPLIT))
    return loss[0, 0]
